# baseline (speedup 1.0000x reference)
import functools

import jax
import jax.numpy as jnp
from jax import lax
from jax.experimental import pallas as pl
from jax.experimental.pallas import tpu as pltpu
from jax.experimental.pallas import tpu_sc as plsc

B = 16384
N_LAYERS = 8
VOCAB = 100000
DIM = 64

NUM_CORES = 2
NUM_SUBCORES = 16
NUM_WORKERS = NUM_CORES * NUM_SUBCORES
TOK_PER_WORKER = B // NUM_WORKERS
CHUNK = 64
NUM_CHUNKS = TOK_PER_WORKER // CHUNK
LANES = 16

_mesh = plsc.VectorSubcoreMesh(core_axis_name="c", subcore_axis_name="s")


@functools.partial(
    pl.kernel,
    out_type=jax.ShapeDtypeStruct((B, DIM), jnp.float32),
    mesh=_mesh,
    compiler_params=pltpu.CompilerParams(
        needs_layout_passes=False, use_tc_tiling_on_sc=False
    ),
    scratch_types=[
        pltpu.VMEM((CHUNK * N_LAYERS,), jnp.int32),
        pltpu.VMEM((N_LAYERS, CHUNK), jnp.int32),
        pltpu.VMEM((CHUNK, DIM), jnp.float32),
        pltpu.SemaphoreType.DMA,
    ],
)
def _res_emb(x_hbm, emb_hbm, out_hbm, xv, fi, acc, sem):
    wid = lax.axis_index("s") * NUM_CORES + lax.axis_index("c")
    base = wid * TOK_PER_WORKER
    lane_iota = lax.iota(jnp.int32, LANES)
    zeros = jnp.zeros((LANES,), jnp.float32)

    @pl.loop(0, NUM_CHUNKS)
    def _chunk(ci):
        tok = base + ci * CHUNK
        pltpu.sync_copy(x_hbm.at[pl.ds(tok * N_LAYERS, CHUNK * N_LAYERS)], xv)
        for l in range(N_LAYERS):
            for v in range(CHUNK // LANES):
                flat = (lane_iota + (v * LANES)) * N_LAYERS + l
                g = plsc.load_gather(xv, [flat])
                fi[l, pl.ds(v * LANES, LANES)] = g + l * VOCAB
        for t in range(CHUNK):
            for s in range(DIM // LANES):
                acc[t, pl.ds(s * LANES, LANES)] = zeros
        copies = [
            pltpu.async_copy(emb_hbm.at[fi.at[l]], acc, sem, add=True)
            for l in range(N_LAYERS)
        ]
        for c in copies:
            c.wait()
        pltpu.sync_copy(acc, out_hbm.at[pl.ds(tok, CHUNK)])


def kernel(x, emb):
    x_flat = x.astype(jnp.int32).reshape(B * N_LAYERS)
    emb_flat = emb.reshape(N_LAYERS * VOCAB, DIM)
    return _res_emb(x_flat, emb_flat)


# speedup vs baseline: 1.0749x; 1.0749x over previous
import functools

import jax
import jax.numpy as jnp
from jax import lax
from jax.experimental import pallas as pl
from jax.experimental.pallas import tpu as pltpu
from jax.experimental.pallas import tpu_sc as plsc

B = 16384
N_LAYERS = 8
VOCAB = 100000
DIM = 64
ROWS = N_LAYERS * VOCAB
PAD = 128

NUM_CORES = 2
NUM_SUBCORES = 16
NUM_WORKERS = NUM_CORES * NUM_SUBCORES
TOK_PER_WORKER = B // NUM_WORKERS
CHUNK = 64
NUM_CHUNKS = TOK_PER_WORKER // CHUNK
LANES = 16

STAGE_BM = 8000
STAGE_GRID = ROWS // STAGE_BM

_mesh = plsc.VectorSubcoreMesh(core_axis_name="c", subcore_axis_name="s")
_sc_params = pltpu.CompilerParams(
    needs_layout_passes=False, use_tc_tiling_on_sc=True
)


def _stage_body(i_ref, o_ref):
    x = i_ref[...]
    o_ref[...] = jnp.concatenate([x, x], axis=1)


_stage = pl.pallas_call(
    _stage_body,
    grid=(STAGE_GRID,),
    in_specs=[pl.BlockSpec((STAGE_BM, DIM), lambda i: (i, 0))],
    out_specs=pl.BlockSpec((STAGE_BM, PAD), lambda i: (i, 0)),
    out_shape=jax.ShapeDtypeStruct((ROWS, PAD), jnp.float32),
)


@functools.partial(
    pl.kernel,
    out_type=jax.ShapeDtypeStruct((B, PAD), jnp.float32),
    mesh=_mesh,
    compiler_params=_sc_params,
    scratch_types=[
        pltpu.VMEM((CHUNK * N_LAYERS,), jnp.int32),
        pltpu.VMEM((N_LAYERS, CHUNK), jnp.int32),
        pltpu.VMEM((CHUNK, PAD), jnp.float32),
        pltpu.SemaphoreType.DMA,
    ],
)
def _lookup(x_hbm, tab_hbm, out_hbm, xv, fi, acc, sem):
    base = (lax.axis_index("s") * NUM_CORES + lax.axis_index("c")) * TOK_PER_WORKER
    lane_iota = lax.iota(jnp.int32, LANES)
    zeros = jnp.zeros((LANES,), jnp.float32)

    @pl.loop(0, NUM_CHUNKS)
    def _chunk(ci):
        tok = base + ci * CHUNK
        pltpu.sync_copy(x_hbm.at[pl.ds(tok * N_LAYERS, CHUNK * N_LAYERS)], xv)
        for l in range(N_LAYERS):
            for v in range(CHUNK // LANES):
                flat = (lane_iota + (v * LANES)) * N_LAYERS + l
                g = plsc.load_gather(xv, [flat])
                fi[l, pl.ds(v * LANES, LANES)] = g + l * VOCAB
        for t in range(CHUNK):
            for s in range(PAD // LANES):
                acc[t, pl.ds(s * LANES, LANES)] = zeros
        copies = [
            pltpu.async_copy(tab_hbm.at[fi.at[l]], acc, sem, add=True)
            for l in range(N_LAYERS)
        ]
        for c in copies:
            c.wait()
        pltpu.sync_copy(acc, out_hbm.at[pl.ds(tok, CHUNK)])


def kernel(x, emb):
    x_flat = x.astype(jnp.int32).reshape(B * N_LAYERS)
    emb_flat = emb.reshape(ROWS, DIM)
    padded = _stage(emb_flat)
    wide = _lookup(x_flat, padded)
    return wide[:, :DIM]


# speedup vs baseline: 1.2008x; 1.1171x over previous
import functools

import jax
import jax.numpy as jnp
from jax import lax
from jax.experimental import pallas as pl
from jax.experimental.pallas import tpu as pltpu
from jax.experimental.pallas import tpu_sc as plsc

B = 16384
N_LAYERS = 8
VOCAB = 100000
DIM = 64
ROWS = N_LAYERS * VOCAB
PAD = 128

NUM_CORES = 2
NUM_SUBCORES = 16
NUM_WORKERS = NUM_CORES * NUM_SUBCORES
TOK_PER_WORKER = B // NUM_WORKERS
CHUNK = 64
NUM_CHUNKS = TOK_PER_WORKER // CHUNK
LANES = 16
IDX = CHUNK * N_LAYERS

STAGE_BV = 6272
STAGE_GRID = 8
HALF = STAGE_BV * STAGE_GRID

_mesh = plsc.VectorSubcoreMesh(core_axis_name="c", subcore_axis_name="s")
_sc_params = pltpu.CompilerParams(
    needs_layout_passes=False, use_tc_tiling_on_sc=True
)


def _stage_body(lo_ref, hi_ref, o_ref):
    o_ref[0, :, :DIM] = lo_ref[0].T
    o_ref[0, :, DIM:] = hi_ref[0].T


_stage = pl.pallas_call(
    _stage_body,
    grid=(N_LAYERS, STAGE_GRID),
    in_specs=[
        pl.BlockSpec((1, DIM, STAGE_BV), lambda l, i: (l, 0, i)),
        pl.BlockSpec((1, DIM, STAGE_BV), lambda l, i: (l, 0, i + STAGE_GRID)),
    ],
    out_specs=pl.BlockSpec((1, STAGE_BV, PAD), lambda l, i: (l, i, 0)),
    out_shape=jax.ShapeDtypeStruct((N_LAYERS, HALF, PAD), jnp.float32),
)


@functools.partial(
    pl.kernel,
    out_type=jax.ShapeDtypeStruct((B, PAD), jnp.float32),
    mesh=_mesh,
    compiler_params=_sc_params,
    scratch_types=[
        pltpu.VMEM((IDX,), jnp.int32),
        pltpu.VMEM((4, IDX // 4), jnp.int32),
        pltpu.VMEM((IDX, PAD), jnp.float32),
        pltpu.VMEM((CHUNK, PAD), jnp.float32),
        pltpu.SemaphoreType.DMA,
    ],
)
def _lookup(x_hbm, tab_hbm, out_hbm, xv, fi, rows8, acc, sem):
    base = (lax.axis_index("s") * NUM_CORES + lax.axis_index("c")) * TOK_PER_WORKER
    lane_iota = lax.iota(jnp.int32, LANES)
    offs = (lane_iota & 7) * HALF
    zeros = jnp.zeros((LANES,), jnp.float32)

    @pl.loop(0, NUM_CHUNKS)
    def _chunk(ci):
        tok = base + ci * CHUNK
        pltpu.sync_copy(x_hbm.at[pl.ds(tok * N_LAYERS, IDX)], xv)
        for r in range(4):
            for v in range(8):
                j0 = r * 128 + v * LANES
                vals = xv[pl.ds(j0, LANES)]
                hi = (vals >= HALF).astype(jnp.int32)
                fi[r, pl.ds(v * LANES, LANES)] = vals - hi * HALF + offs
        copies = [
            pltpu.async_copy(
                tab_hbm.at[fi.at[r]], rows8.at[pl.ds(r * 128, 128)], sem
            )
            for r in range(4)
        ]
        for c in copies:
            c.wait()
        @pl.loop(0, CHUNK // LANES)
        def _grp(g):
            rows = []
            cols = []
            for l in range(N_LAYERS):
                r_l = g * (LANES * N_LAYERS) + lane_iota * N_LAYERS + l
                hi = (plsc.load_gather(xv, [r_l]) >= HALF).astype(jnp.int32)
                rows.append(r_l)
                cols.append(hi * DIM)
            tok16 = lane_iota + g * LANES
            for d in range(DIM):
                a = plsc.load_gather(rows8, [rows[0], cols[0] + d])
                for l in range(1, N_LAYERS):
                    a = a + plsc.load_gather(rows8, [rows[l], cols[l] + d])
                plsc.store_scatter(acc, [tok16, jnp.full((LANES,), d, jnp.int32)], a)

        pltpu.sync_copy(acc, out_hbm.at[pl.ds(tok, CHUNK)])


def kernel(x, emb):
    x_flat = x.astype(jnp.int32).reshape(B * N_LAYERS)
    emb_t = jnp.transpose(emb, (0, 2, 1))
    packed = _stage(emb_t, emb_t).reshape(N_LAYERS * HALF, PAD)
    wide = _lookup(x_flat, packed)
    return wide[:, :DIM]


# speedup vs baseline: 1.7245x; 1.4361x over previous
import functools

import jax
import jax.numpy as jnp
from jax import lax
from jax.experimental import pallas as pl
from jax.experimental.pallas import tpu as pltpu
from jax.experimental.pallas import tpu_sc as plsc

B = 16384
N_LAYERS = 8
VOCAB = 100000
DIM = 64
ROWS = N_LAYERS * VOCAB
PAD = 128

NUM_CORES = 2
NUM_SUBCORES = 16
NUM_WORKERS = NUM_CORES * NUM_SUBCORES
TOK_PER_WORKER = B // NUM_WORKERS
CHUNK = 128
NUM_CHUNKS = TOK_PER_WORKER // CHUNK
LANES = 16

STAGE_BV = 12800
STAGE_GRID = -(-VOCAB // STAGE_BV)

_mesh = plsc.VectorSubcoreMesh(core_axis_name="c", subcore_axis_name="s")
_sc_params = pltpu.CompilerParams(
    needs_layout_passes=False, use_tc_tiling_on_sc=True
)


def _stage_body(i_ref, o_ref):
    x = i_ref[0]
    xt = x.T
    o_ref[0] = jnp.concatenate([xt, xt], axis=1)


_stage = pl.pallas_call(
    _stage_body,
    grid=(8, STAGE_GRID),
    in_specs=[pl.BlockSpec((1, DIM, STAGE_BV), lambda l, i: (l, 0, i))],
    out_specs=pl.BlockSpec((1, STAGE_BV, PAD), lambda l, i: (l, i, 0)),
    out_shape=jax.ShapeDtypeStruct((N_LAYERS, VOCAB, PAD), jnp.float32),
)


@functools.partial(
    pl.kernel,
    out_type=jax.ShapeDtypeStruct((B, PAD), jnp.float32),
    mesh=_mesh,
    compiler_params=_sc_params,
    scratch_types=[
        pltpu.VMEM((N_LAYERS, CHUNK), jnp.int32),
        pltpu.VMEM((N_LAYERS, CHUNK), jnp.int32),
        pltpu.VMEM((CHUNK, PAD), jnp.float32),
        pltpu.SemaphoreType.DMA,
    ],
)
def _lookup(x_hbm, tab_hbm, out_hbm, xv, fi, acc, sem):
    base = (lax.axis_index("s") * NUM_CORES + lax.axis_index("c")) * TOK_PER_WORKER
    lane_iota = lax.iota(jnp.int32, LANES)
    zeros = jnp.zeros((LANES,), jnp.float32)

    @pl.loop(0, NUM_CHUNKS)
    def _chunk(ci):
        tok = base + ci * CHUNK
        pltpu.sync_copy(x_hbm.at[:, pl.ds(tok, CHUNK)], xv)
        for l in range(N_LAYERS):
            for v in range(CHUNK // LANES):
                sl = pl.ds(v * LANES, LANES)
                fi[l, sl] = xv[l, sl] + l * VOCAB
        for t in range(CHUNK):
            for s in range(PAD // LANES):
                acc[t, pl.ds(s * LANES, LANES)] = zeros
        copies = [
            pltpu.async_copy(tab_hbm.at[fi.at[l]], acc, sem, add=True)
            for l in range(N_LAYERS)
        ]
        for c in copies:
            c.wait()
        pltpu.sync_copy(acc, out_hbm.at[pl.ds(tok, CHUNK)])


def kernel(x, emb):
    x_t = x.astype(jnp.int32).T
    emb_t = jnp.transpose(emb, (0, 2, 1))
    padded = _stage(emb_t).reshape(ROWS, PAD)
    wide = _lookup(x_t, padded)
    return wide[:, :DIM]
